# baseline (speedup 1.0000x reference)
import functools

import jax
import jax.numpy as jnp
from jax import lax
from jax.experimental import pallas as pl
from jax.experimental.pallas import tpu as pltpu
from jax.experimental.pallas import tpu_sc as plsc

L = 16
EPS = 1e-12


def _rsqrt(x):
    i = lax.bitcast_convert_type(x, jnp.int32)
    i = jnp.int32(0x5F3759DF) - lax.shift_right_arithmetic(i, 1)
    y = lax.bitcast_convert_type(i, jnp.float32)
    for _ in range(3):
        y = y * (1.5 - 0.5 * x * y * y)
    return y


def _perm(v, perm):
    return v.at[perm].get(mode="promise_in_bounds", unique_indices=True)


def _tree_lanesums(vs):
    lanes = lax.iota(jnp.int32, L)
    k = 1
    while len(vs) > 1:
        perm = jnp.bitwise_xor(lanes, k)
        m = (lanes & k) == 0
        nxt = []
        for i in range(len(vs) // 2):
            a, b = vs[2 * i], vs[2 * i + 1]
            d = jnp.where(m, a, b)
            e = jnp.where(m, b, a)
            nxt.append(d + _perm(e, perm))
        vs = nxt
        k *= 2
    return vs[0]


@functools.lru_cache(maxsize=None)
def _build(B, S, H, NC, NS):
    NW = NC * NS
    SW = S // NW
    G = H // L
    mesh = plsc.VectorSubcoreMesh(core_axis_name="c", subcore_axis_name="s")

    @functools.partial(
        pl.kernel,
        out_type=jax.ShapeDtypeStruct((B, S, H), jnp.float32),
        mesh=mesh,
        scratch_types=[
            pltpu.VMEM((B * SW,), jnp.int32),
            pltpu.VMEM((B * SW,), jnp.int32),
            pltpu.VMEM((2 * SW, H), jnp.float32),
            pltpu.VMEM((2, H), jnp.float32),
            pltpu.VMEM((2, SW, H), jnp.float32),
            pltpu.VMEM((2, SW, H), jnp.float32),
            pltpu.VMEM((H,), jnp.float32),
            pltpu.VMEM((H,), jnp.float32),
            pltpu.SemaphoreType.DMA((2,)),
            pltpu.SemaphoreType.DMA((2,)),
        ],
    )
    def kern(ids_h, tt_h, word_h, pos_h, type_h, lnw_h, lnb_h, out_h,
             idsw_v, ttw_v, p23_v, type_v, grow_v, obuf_v, lnw_v, lnb_v,
             gsem, osem):
        wid = lax.axis_index("s") * NC + lax.axis_index("c")
        s0 = wid * SW

        pltpu.sync_copy(ids_h.at[wid], idsw_v)
        pltpu.sync_copy(tt_h.at[wid], ttw_v)
        pltpu.sync_copy(pos_h.at[pl.ds(s0, SW)], p23_v.at[pl.ds(0, SW)])
        pltpu.sync_copy(pos_h.at[pl.ds(s0, SW)], p23_v.at[pl.ds(SW, SW)])
        pltpu.sync_copy(type_h, type_v)
        pltpu.sync_copy(lnw_h, lnw_v)
        pltpu.sync_copy(lnb_h, lnb_v)

        def prep(i):
            j = i // G
            g = i % G
            sl = pl.ds(g * L, L)
            p23_v[j, sl] = p23_v[j, sl] + type_v[0, sl]
            p23_v[SW + j, sl] = p23_v[SW + j, sl] + type_v[1, sl]
        plsc.parallel_loop(0, SW * G, unroll=8)(prep)

        zero = jnp.zeros((L,), jnp.float32)
        inv_h = jnp.float32(1.0 / H)

        def gather_copy(b, h):
            iv = idsw_v[pl.ds(b * SW, SW)]
            return pltpu.make_async_copy(
                word_h.at[iv], grow_v.at[h], gsem.at[h])

        def out_copy(b, h):
            return pltpu.make_async_copy(
                obuf_v.at[h], out_h.at[b, pl.ds(s0, SW)], osem.at[h])

        def fill_copy(h):
            return pltpu.make_async_copy(
                word_h.at[pl.ds(0, SW)], obuf_v.at[h], osem.at[h])

        def compute(b, h):
            ttrow = ttw_v[pl.ds(b * SW, SW)]
            svecs = []
            qvecs = []
            for j in range(SW):
                rbias = ttrow[j] * SW + j

                def p1(i, carry):
                    accs = list(carry)
                    for t in range(8):
                        sl = pl.ds((8 * i + t) * L, L)
                        x = grow_v[h, j, sl] + p23_v[rbias, sl]
                        obuf_v[h, j, sl] = x
                        k = t % 4
                        accs[2 * k] = accs[2 * k] + x
                        accs[2 * k + 1] = accs[2 * k + 1] + x * x
                    return tuple(accs)

                accs = plsc.parallel_loop(
                    0, G // 8, carry=(zero,) * 8)(p1)
                svecs.append(accs[0] + accs[2] + (accs[4] + accs[6]))
                qvecs.append(accs[1] + accs[3] + (accs[5] + accs[7]))

            m_vec = _tree_lanesums(svecs) * inv_h
            var_vec = _tree_lanesums(qvecs) * inv_h - m_vec * m_vec
            rstd_vec = _rsqrt(var_vec + EPS)
            means = [m_vec[j] for j in range(SW)]
            rstds = [rstd_vec[j] for j in range(SW)]

            def p2(i):
                for t in range(2):
                    sl = pl.ds((2 * i + t) * L, L)
                    lw = lnw_v[sl]
                    lb = lnb_v[sl]
                    for j in range(SW):
                        x = obuf_v[h, j, sl]
                        obuf_v[h, j, sl] = (x - means[j]) * rstds[j] * lw + lb

            plsc.parallel_loop(0, G // 2)(p2)

        fill_copy(0).start()
        fill_copy(1).start()
        gather_copy(0, 0).start()
        gather_copy(1, 1).start()

        def outer(ko, _):
            c0 = 2 * ko
            for h in range(2):
                c = c0 + h
                gather_copy(c, h).wait()
                out_copy(jnp.maximum(c - 2, 0), h).wait()
                compute(c, h)
                gather_copy(jnp.minimum(c + 2, B - 1), h).start()
                out_copy(c, h).start()
            return 0

        lax.fori_loop(0, B // 2, outer, 0)

        gather_copy(B - 1, 0).wait()
        gather_copy(B - 1, 1).wait()
        out_copy(B - 2, 0).wait()
        out_copy(B - 1, 1).wait()

    return kern


def kernel(input_ids, token_type_ids, word_emb, pos_emb, type_emb, ln_w, ln_b):
    B, S = input_ids.shape
    _, H = word_emb.shape
    info = plsc.get_sparse_core_info()
    NW = info.num_cores * info.num_subcores
    SW = S // NW
    ids_r = (input_ids.astype(jnp.int32)
             .reshape(B, NW, SW).transpose(1, 0, 2).reshape(NW, B * SW))
    tt_r = (token_type_ids.astype(jnp.int32)
            .reshape(B, NW, SW).transpose(1, 0, 2).reshape(NW, B * SW))
    kern = _build(B, S, H, info.num_cores, info.num_subcores)
    return kern(ids_r, tt_r, word_emb, pos_emb, type_emb, ln_w, ln_b)


# speedup vs baseline: 1.3186x; 1.3186x over previous
import functools

import jax
import jax.numpy as jnp
from jax import lax
from jax.experimental import pallas as pl
from jax.experimental.pallas import tpu as pltpu
from jax.experimental.pallas import tpu_sc as plsc

L = 16
EPS = 1e-12


def _rsqrt(x):
    i = lax.bitcast_convert_type(x, jnp.int32)
    i = jnp.int32(0x5F3759DF) - lax.shift_right_arithmetic(i, 1)
    y = lax.bitcast_convert_type(i, jnp.float32)
    for _ in range(3):
        y = y * (1.5 - 0.5 * x * y * y)
    return y


def _perm(v, perm):
    return v.at[perm].get(mode="promise_in_bounds", unique_indices=True)


def _tree_lanesums(vs):
    lanes = lax.iota(jnp.int32, L)
    k = 1
    while len(vs) > 1:
        perm = jnp.bitwise_xor(lanes, k)
        m = (lanes & k) == 0
        nxt = []
        for i in range(len(vs) // 2):
            a, b = vs[2 * i], vs[2 * i + 1]
            d = jnp.where(m, a, b)
            e = jnp.where(m, b, a)
            nxt.append(d + _perm(e, perm))
        vs = nxt
        k *= 2
    return vs[0]


@functools.lru_cache(maxsize=None)
def _build(B, S, H, NC, NS):
    NW = NC * NS
    SW = S // NW
    G = H // L
    mesh = plsc.VectorSubcoreMesh(core_axis_name="c", subcore_axis_name="s")

    @functools.partial(
        pl.kernel,
        out_type=jax.ShapeDtypeStruct((B, S, H), jnp.float32),
        mesh=mesh,
        scratch_types=[
            pltpu.VMEM((B * SW,), jnp.int32),
            pltpu.VMEM((B * SW,), jnp.int32),
            pltpu.VMEM((2 * SW, H), jnp.float32),
            pltpu.VMEM((2, H), jnp.float32),
            pltpu.VMEM((2, SW, H), jnp.float32),
            pltpu.VMEM((2, SW, H), jnp.float32),
            pltpu.VMEM((H,), jnp.float32),
            pltpu.VMEM((H,), jnp.float32),
            pltpu.SemaphoreType.DMA((2,)),
            pltpu.SemaphoreType.DMA((2,)),
        ],
    )
    def kern(ids_h, tt_h, word_h, pos_h, type_h, lnw_h, lnb_h, out_h,
             idsw_v, ttw_v, p23_v, type_v, grow_v, obuf_v, lnw_v, lnb_v,
             gsem, osem):
        wid = lax.axis_index("s") * NC + lax.axis_index("c")
        s0 = wid * SW

        pltpu.sync_copy(ids_h.at[wid], idsw_v)
        pltpu.sync_copy(tt_h.at[wid], ttw_v)
        pltpu.sync_copy(pos_h.at[pl.ds(s0, SW)], p23_v.at[pl.ds(0, SW)])
        pltpu.sync_copy(pos_h.at[pl.ds(s0, SW)], p23_v.at[pl.ds(SW, SW)])
        pltpu.sync_copy(type_h, type_v)
        pltpu.sync_copy(lnw_h, lnw_v)
        pltpu.sync_copy(lnb_h, lnb_v)

        def prep(i):
            j = i // G
            g = i % G
            sl = pl.ds(g * L, L)
            p23_v[j, sl] = p23_v[j, sl] + type_v[0, sl]
            p23_v[SW + j, sl] = p23_v[SW + j, sl] + type_v[1, sl]
        plsc.parallel_loop(0, SW * G, unroll=8)(prep)

        zero = jnp.zeros((L,), jnp.float32)
        inv_h = jnp.float32(1.0 / H)

        def gather_copy(b, h):
            iv = idsw_v[pl.ds(b * SW, SW)]
            return pltpu.make_async_copy(
                word_h.at[iv], grow_v.at[h], gsem.at[h])

        def out_copy(b, h):
            return pltpu.make_async_copy(
                obuf_v.at[h], out_h.at[b, pl.ds(s0, SW)], osem.at[h])

        def fill_copy(h):
            return pltpu.make_async_copy(
                word_h.at[pl.ds(0, SW)], obuf_v.at[h], osem.at[h])

        TH = SW // 2

        def compute(b, h):
            ttrow = ttw_v[pl.ds(b * SW, SW)]
            rbias = [ttrow[j] * SW + j for j in range(SW)]
            svecs = [None] * SW
            qvecs = [None] * SW
            for half in range(2):
                toks = list(range(half * TH, half * TH + TH))

                def p1(g, carry):
                    accs = list(carry)
                    sl = pl.ds(g * L, L)
                    for t, j in enumerate(toks):
                        x = grow_v[h, j, sl] + p23_v[rbias[j], sl]
                        obuf_v[h, j, sl] = x
                        accs[2 * t] = accs[2 * t] + x
                        accs[2 * t + 1] = accs[2 * t + 1] + x * x
                    return tuple(accs)

                accs = plsc.parallel_loop(
                    0, G, carry=(zero,) * (2 * TH))(p1)
                for t, j in enumerate(toks):
                    svecs[j] = accs[2 * t]
                    qvecs[j] = accs[2 * t + 1]

            m_vec = _tree_lanesums(svecs) * inv_h
            var_vec = _tree_lanesums(qvecs) * inv_h - m_vec * m_vec
            rstd_vec = _rsqrt(var_vec + EPS)
            means = [m_vec[j] for j in range(SW)]
            rstds = [rstd_vec[j] for j in range(SW)]

            def p2(g):
                sl = pl.ds(g * L, L)
                lw = lnw_v[sl]
                lb = lnb_v[sl]
                for j in range(SW):
                    x = obuf_v[h, j, sl]
                    obuf_v[h, j, sl] = (x - means[j]) * rstds[j] * lw + lb

            plsc.parallel_loop(0, G, unroll=2)(p2)

        fill_copy(0).start()
        fill_copy(1).start()
        gather_copy(0, 0).start()
        gather_copy(1, 1).start()

        def outer(ko, _):
            c0 = 2 * ko
            for h in range(2):
                c = c0 + h
                gather_copy(c, h).wait()
                out_copy(jnp.maximum(c - 2, 0), h).wait()
                compute(c, h)
                gather_copy(jnp.minimum(c + 2, B - 1), h).start()
                out_copy(c, h).start()
            return 0

        lax.fori_loop(0, B // 2, outer, 0)

        gather_copy(B - 1, 0).wait()
        gather_copy(B - 1, 1).wait()
        out_copy(B - 2, 0).wait()
        out_copy(B - 1, 1).wait()

    return kern


def kernel(input_ids, token_type_ids, word_emb, pos_emb, type_emb, ln_w, ln_b):
    B, S = input_ids.shape
    _, H = word_emb.shape
    info = plsc.get_sparse_core_info()
    NW = info.num_cores * info.num_subcores
    SW = S // NW
    ids_r = (input_ids.astype(jnp.int32)
             .reshape(B, NW, SW).transpose(1, 0, 2).reshape(NW, B * SW))
    tt_r = (token_type_ids.astype(jnp.int32)
            .reshape(B, NW, SW).transpose(1, 0, 2).reshape(NW, B * SW))
    kern = _build(B, S, H, info.num_cores, info.num_subcores)
    return kern(ids_r, tt_r, word_emb, pos_emb, type_emb, ln_w, ln_b)


# speedup vs baseline: 2.6245x; 1.9903x over previous
import functools

import jax
import jax.numpy as jnp
from jax import lax
from jax.experimental import pallas as pl
from jax.experimental.pallas import tpu as pltpu
from jax.experimental.pallas import tpu_sc as plsc

L = 16
EPS = 1e-12


def _rsqrt(x):
    i = lax.bitcast_convert_type(x, jnp.int32)
    i = jnp.int32(0x5F3759DF) - lax.shift_right_arithmetic(i, 1)
    y = lax.bitcast_convert_type(i, jnp.float32)
    for _ in range(3):
        y = y * (1.5 - 0.5 * x * y * y)
    return y


def _perm(v, perm):
    return v.at[perm].get(mode="promise_in_bounds", unique_indices=True)


def _tree_lanesums8(vs):
    lanes = lax.iota(jnp.int32, L)
    k = 1
    while len(vs) > 1:
        perm = jnp.bitwise_xor(lanes, k)
        m = (lanes & k) == 0
        nxt = []
        for i in range(len(vs) // 2):
            a, b = vs[2 * i], vs[2 * i + 1]
            d = jnp.where(m, a, b)
            e = jnp.where(m, b, a)
            nxt.append(d + _perm(e, perm))
        vs = nxt
        k *= 2
    t = vs[0]
    return t + _perm(t, jnp.bitwise_xor(lanes, 8))


@functools.lru_cache(maxsize=None)
def _build(B, S, H, NC, NS):
    NW = NC * NS
    SW = S // NW
    G = H // L
    mesh = plsc.VectorSubcoreMesh(core_axis_name="c", subcore_axis_name="s")

    @functools.partial(
        pl.kernel,
        out_type=jax.ShapeDtypeStruct((B, S, H), jnp.float32),
        mesh=mesh,
        scratch_types=[
            pltpu.VMEM((B * SW,), jnp.int32),
            pltpu.VMEM((B * SW,), jnp.int32),
            pltpu.VMEM((2 * SW, H), jnp.float32),
            pltpu.VMEM((2, H), jnp.float32),
            pltpu.VMEM((2, SW, H), jnp.float32),
            pltpu.VMEM((2, SW, H), jnp.float32),
            pltpu.VMEM((H,), jnp.float32),
            pltpu.VMEM((H,), jnp.float32),
            pltpu.SemaphoreType.DMA((2,)),
            pltpu.SemaphoreType.DMA((2,)),
        ],
    )
    def kern(ids_h, tt_h, word_h, pos_h, type_h, lnw_h, lnb_h, out_h,
             idsw_v, ttw_v, p23_v, type_v, grow_v, obuf_v, lnw_v, lnb_v,
             gsem, osem):
        wid = lax.axis_index("s") * NC + lax.axis_index("c")
        s0 = wid * SW

        pltpu.sync_copy(ids_h.at[wid], idsw_v)
        pltpu.sync_copy(tt_h.at[wid], ttw_v)
        pltpu.sync_copy(pos_h.at[pl.ds(s0, SW)], p23_v.at[pl.ds(0, SW)])
        pltpu.sync_copy(pos_h.at[pl.ds(s0, SW)], p23_v.at[pl.ds(SW, SW)])
        pltpu.sync_copy(type_h, type_v)
        pltpu.sync_copy(lnw_h, lnw_v)
        pltpu.sync_copy(lnb_h, lnb_v)

        def prep(i):
            j = i // G
            g = i % G
            sl = pl.ds(g * L, L)
            p23_v[j, sl] = p23_v[j, sl] + type_v[0, sl]
            p23_v[SW + j, sl] = p23_v[SW + j, sl] + type_v[1, sl]
        plsc.parallel_loop(0, SW * G, unroll=8)(prep)

        zero = jnp.zeros((L,), jnp.float32)
        inv_h = jnp.float32(1.0 / H)

        def gather_copy(b, h):
            iv = idsw_v[pl.ds(b * SW, SW)]
            return pltpu.make_async_copy(
                word_h.at[iv], grow_v.at[h], gsem.at[h])

        def out_copy(b, h):
            return pltpu.make_async_copy(
                obuf_v.at[h], out_h.at[b, pl.ds(s0, SW)], osem.at[h])

        def fill_copy(h):
            return pltpu.make_async_copy(
                word_h.at[pl.ds(0, SW)], obuf_v.at[h], osem.at[h])

        TH = SW // 2
        lanes_lt8 = lax.iota(jnp.int32, L) < 8

        def compute(b, h):
            ttrow = ttw_v[pl.ds(b * SW, SW)]
            rbias = [ttrow[j] * SW + j for j in range(SW)]
            strees = []
            qtrees = []
            for half in range(2):
                toks = list(range(half * TH, half * TH + TH))

                def p1(g, carry):
                    accs = list(carry)
                    sl = pl.ds(g * L, L)
                    xs = [grow_v[h, j, sl] + p23_v[rbias[j], sl]
                          for j in toks]
                    for t, j in enumerate(toks):
                        obuf_v[h, j, sl] = xs[t]
                        accs[2 * t] = accs[2 * t] + xs[t]
                        accs[2 * t + 1] = accs[2 * t + 1] + xs[t] * xs[t]
                    return tuple(accs)

                accs = plsc.parallel_loop(
                    0, G, carry=(zero,) * (2 * TH))(p1)
                strees.append(_tree_lanesums8(accs[0::2]))
                qtrees.append(_tree_lanesums8(accs[1::2]))

            s_all = jnp.where(lanes_lt8, strees[0], strees[1])
            q_all = jnp.where(lanes_lt8, qtrees[0], qtrees[1])
            m_vec = s_all * inv_h
            var_vec = q_all * inv_h - m_vec * m_vec
            rstd_vec = _rsqrt(var_vec + EPS)

            for tb in range(SW // 4):
                toks = list(range(4 * tb, 4 * tb + 4))
                ms = [m_vec[j] for j in toks]
                rs = [rstd_vec[j] for j in toks]

                def p2(i):
                    for u in range(2):
                        sl = pl.ds((2 * i + u) * L, L)
                        lw = lnw_v[sl]
                        lb = lnb_v[sl]
                        xs = [obuf_v[h, j, sl] for j in toks]
                        for t, j in enumerate(toks):
                            obuf_v[h, j, sl] = ((xs[t] - ms[t]) * rs[t]
                                                * lw + lb)

                plsc.parallel_loop(0, G // 2)(p2)

        fill_copy(0).start()
        fill_copy(1).start()
        gather_copy(0, 0).start()
        gather_copy(1, 1).start()

        def outer(ko, _):
            c0 = 2 * ko
            for h in range(2):
                c = c0 + h
                gather_copy(c, h).wait()
                out_copy(jnp.maximum(c - 2, 0), h).wait()
                compute(c, h)
                gather_copy(jnp.minimum(c + 2, B - 1), h).start()
                out_copy(c, h).start()
            return 0

        lax.fori_loop(0, B // 2, outer, 0)

        gather_copy(B - 1, 0).wait()
        gather_copy(B - 1, 1).wait()
        out_copy(B - 2, 0).wait()
        out_copy(B - 1, 1).wait()

    return kern


def kernel(input_ids, token_type_ids, word_emb, pos_emb, type_emb, ln_w, ln_b):
    B, S = input_ids.shape
    _, H = word_emb.shape
    info = plsc.get_sparse_core_info()
    NW = info.num_cores * info.num_subcores
    SW = S // NW
    ids_r = (input_ids.astype(jnp.int32)
             .reshape(B, NW, SW).transpose(1, 0, 2).reshape(NW, B * SW))
    tt_r = (token_type_ids.astype(jnp.int32)
            .reshape(B, NW, SW).transpose(1, 0, 2).reshape(NW, B * SW))
    kern = _build(B, S, H, info.num_cores, info.num_subcores)
    return kern(ids_r, tt_r, word_emb, pos_emb, type_emb, ln_w, ln_b)


# speedup vs baseline: 2.7014x; 1.0293x over previous
import functools

import jax
import jax.numpy as jnp
from jax import lax
from jax.experimental import pallas as pl
from jax.experimental.pallas import tpu as pltpu
from jax.experimental.pallas import tpu_sc as plsc

L = 16
EPS = 1e-12


def _rsqrt(x):
    i = lax.bitcast_convert_type(x, jnp.int32)
    i = jnp.int32(0x5F3759DF) - lax.shift_right_arithmetic(i, 1)
    y = lax.bitcast_convert_type(i, jnp.float32)
    for _ in range(3):
        y = y * (1.5 - 0.5 * x * y * y)
    return y


def _perm(v, perm):
    return v.at[perm].get(mode="promise_in_bounds", unique_indices=True)


def _tree_lanesums8(vs):
    lanes = lax.iota(jnp.int32, L)
    k = 1
    while len(vs) > 1:
        perm = jnp.bitwise_xor(lanes, k)
        m = (lanes & k) == 0
        nxt = []
        for i in range(len(vs) // 2):
            a, b = vs[2 * i], vs[2 * i + 1]
            d = jnp.where(m, a, b)
            e = jnp.where(m, b, a)
            nxt.append(d + _perm(e, perm))
        vs = nxt
        k *= 2
    t = vs[0]
    return t + _perm(t, jnp.bitwise_xor(lanes, 8))


@functools.lru_cache(maxsize=None)
def _build(B, S, H, NC, NS):
    NW = NC * NS
    SW = S // NW
    G = H // L
    mesh = plsc.VectorSubcoreMesh(core_axis_name="c", subcore_axis_name="s")

    @functools.partial(
        pl.kernel,
        out_type=jax.ShapeDtypeStruct((B, S, H), jnp.float32),
        mesh=mesh,
        scratch_types=[
            pltpu.VMEM((B * SW,), jnp.int32),
            pltpu.VMEM((B * SW,), jnp.int32),
            pltpu.VMEM((2 * SW, H), jnp.float32),
            pltpu.VMEM((2, H), jnp.float32),
            pltpu.VMEM((2, SW, H), jnp.float32),
            pltpu.VMEM((2, SW, H), jnp.float32),
            pltpu.VMEM((H,), jnp.float32),
            pltpu.VMEM((H,), jnp.float32),
            pltpu.SemaphoreType.DMA((2,)),
            pltpu.SemaphoreType.DMA((2,)),
        ],
    )
    def kern(ids_h, tt_h, word_h, pos_h, type_h, lnw_h, lnb_h, out_h,
             idsw_v, ttw_v, p23_v, type_v, grow_v, obuf_v, lnw_v, lnb_v,
             gsem, osem):
        wid = lax.axis_index("s") * NC + lax.axis_index("c")
        s0 = wid * SW

        staging = [
            pltpu.make_async_copy(ids_h.at[wid], idsw_v, gsem.at[0]),
            pltpu.make_async_copy(tt_h.at[wid], ttw_v, gsem.at[0]),
            pltpu.make_async_copy(pos_h.at[pl.ds(s0, SW)],
                                  p23_v.at[pl.ds(0, SW)], gsem.at[0]),
            pltpu.make_async_copy(pos_h.at[pl.ds(s0, SW)],
                                  p23_v.at[pl.ds(SW, SW)], gsem.at[0]),
            pltpu.make_async_copy(type_h, type_v, gsem.at[0]),
            pltpu.make_async_copy(lnw_h, lnw_v, gsem.at[0]),
            pltpu.make_async_copy(lnb_h, lnb_v, gsem.at[0]),
        ]
        for cp in staging:
            cp.start()
        for cp in staging:
            cp.wait()

        def prep(i):
            j = i // G
            g = i % G
            sl = pl.ds(g * L, L)
            p23_v[j, sl] = p23_v[j, sl] + type_v[0, sl]
            p23_v[SW + j, sl] = p23_v[SW + j, sl] + type_v[1, sl]
        plsc.parallel_loop(0, SW * G, unroll=8)(prep)

        zero = jnp.zeros((L,), jnp.float32)
        inv_h = jnp.float32(1.0 / H)

        def gather_copy(b, h):
            iv = idsw_v[pl.ds(b * SW, SW)]
            return pltpu.make_async_copy(
                word_h.at[iv], grow_v.at[h], gsem.at[h])

        def out_copy(b, h):
            return pltpu.make_async_copy(
                obuf_v.at[h], out_h.at[b, pl.ds(s0, SW)], osem.at[h])

        def fill_copy(h):
            return pltpu.make_async_copy(
                word_h.at[pl.ds(0, SW)], obuf_v.at[h], osem.at[h])

        TH = SW // 2
        lanes_lt8 = lax.iota(jnp.int32, L) < 8

        def compute_p1(b, h):
            ttrow = ttw_v[pl.ds(b * SW, SW)]
            rbias = [ttrow[j] * SW + j for j in range(SW)]
            strees = []
            qtrees = []
            for half in range(2):
                toks = list(range(half * TH, half * TH + TH))

                def p1(g, carry):
                    accs = list(carry)
                    sl = pl.ds(g * L, L)
                    xs = [grow_v[h, j, sl] + p23_v[rbias[j], sl]
                          for j in toks]
                    for t, j in enumerate(toks):
                        obuf_v[h, j, sl] = xs[t]
                        accs[2 * t] = accs[2 * t] + xs[t]
                        accs[2 * t + 1] = accs[2 * t + 1] + xs[t] * xs[t]
                    return tuple(accs)

                accs = plsc.parallel_loop(
                    0, G, carry=(zero,) * (2 * TH))(p1)
                strees.append(_tree_lanesums8(accs[0::2]))
                qtrees.append(_tree_lanesums8(accs[1::2]))

            s_all = jnp.where(lanes_lt8, strees[0], strees[1])
            q_all = jnp.where(lanes_lt8, qtrees[0], qtrees[1])
            m_vec = s_all * inv_h
            var_vec = q_all * inv_h - m_vec * m_vec
            rstd_vec = _rsqrt(var_vec + EPS)
            return m_vec, rstd_vec

        def compute_p2(h, m_vec, rstd_vec):
            for tb in range(SW // 4):
                toks = list(range(4 * tb, 4 * tb + 4))
                ms = [m_vec[j] for j in toks]
                rs = [rstd_vec[j] for j in toks]

                def p2(i):
                    for u in range(2):
                        sl = pl.ds((2 * i + u) * L, L)
                        lw = lnw_v[sl]
                        lb = lnb_v[sl]
                        xs = [obuf_v[h, j, sl] for j in toks]
                        for t, j in enumerate(toks):
                            obuf_v[h, j, sl] = ((xs[t] - ms[t]) * rs[t]
                                                * lw + lb)

                plsc.parallel_loop(0, G // 2)(p2)

        fill_copy(0).start()
        fill_copy(1).start()
        gather_copy(0, 0).start()
        gather_copy(1, 1).start()

        def outer(ko, _):
            c0 = 2 * ko
            for h in range(2):
                c = c0 + h
                gather_copy(c, h).wait()
                out_copy(jnp.maximum(c - 2, 0), h).wait()
                mv, rv = compute_p1(c, h)
                gather_copy(jnp.minimum(c + 2, B - 1), h).start()
                compute_p2(h, mv, rv)
                out_copy(c, h).start()
            return 0

        lax.fori_loop(0, B // 2, outer, 0)

        gather_copy(B - 1, 0).wait()
        gather_copy(B - 1, 1).wait()
        out_copy(B - 2, 0).wait()
        out_copy(B - 1, 1).wait()

    return kern


def kernel(input_ids, token_type_ids, word_emb, pos_emb, type_emb, ln_w, ln_b):
    B, S = input_ids.shape
    _, H = word_emb.shape
    info = plsc.get_sparse_core_info()
    NW = info.num_cores * info.num_subcores
    SW = S // NW
    ids_r = (input_ids.astype(jnp.int32)
             .reshape(B, NW, SW).transpose(1, 0, 2).reshape(NW, B * SW))
    tt_r = (token_type_ids.astype(jnp.int32)
            .reshape(B, NW, SW).transpose(1, 0, 2).reshape(NW, B * SW))
    kern = _build(B, S, H, info.num_cores, info.num_subcores)
    return kern(ids_r, tt_r, word_emb, pos_emb, type_emb, ln_w, ln_b)
